# Initial kernel scaffold; baseline (speedup 1.0000x reference)
import jax
import jax.numpy as jnp
from jax.experimental import pallas as pl


def kernel(x, expert, W1, b1, W2, b2):
    raise NotImplementedError("write your pallas kernel here")


# speedup vs baseline: 1.1164x; 1.1164x over previous
import jax
import jax.numpy as jnp
from jax.experimental import pallas as pl
from jax.experimental.pallas import tpu as pltpu

S, D, E, K = 2048, 768, 8, 2
H = 4 * D
TB = 8
BT = S // TB


def _router_kernel(expert_ref, eps_ref, gating_ref):
    z = expert_ref[...]
    eps = eps_ref[...]
    noisy = z + eps * jax.nn.softplus(z)
    idx = jax.lax.broadcasted_iota(jnp.int32, noisy.shape, 1)
    v0 = jnp.max(noisy, axis=1, keepdims=True)
    i0 = jnp.min(jnp.where(noisy == v0, idx, E), axis=1, keepdims=True)
    m0 = idx == i0
    masked = jnp.where(m0, -jnp.inf, noisy)
    v1 = jnp.max(masked, axis=1, keepdims=True)
    i1 = jnp.min(jnp.where(masked == v1, idx, E), axis=1, keepdims=True)
    t = jnp.exp(v1 - v0)
    g0 = 1.0 / (1.0 + t)
    g1 = t / (1.0 + t)
    gating_ref[...] = jnp.where(m0, g0, 0.0) + jnp.where(idx == i1, g1, 0.0)


def _ffn_kernel(gating_ref, x_ref, w1_ref, b1_ref, w2_ref, b2_ref, out_ref):
    e = pl.program_id(0)
    tb = pl.program_id(1)
    x = x_ref[...]
    h = jnp.maximum(
        jnp.dot(x, w1_ref[0], preferred_element_type=jnp.float32) + b1_ref[0], 0.0)
    o = jnp.dot(h, w2_ref[0], preferred_element_type=jnp.float32) + b2_ref[0]
    oh = (jax.lax.broadcasted_iota(jnp.int32, (E, 1), 0) == e).astype(jnp.float32)
    g_blk = gating_ref[pl.ds(tb * BT, BT), :]
    g = jnp.dot(g_blk, oh, preferred_element_type=jnp.float32)
    partial = o * g

    @pl.when(e == 0)
    def _():
        out_ref[pl.ds(tb * BT, BT), :] = partial

    @pl.when(e > 0)
    def _():
        out_ref[pl.ds(tb * BT, BT), :] += partial


def kernel(x, expert, W1, b1, W2, b2):
    eps = jax.random.normal(jax.random.key(42), expert.shape, dtype=jnp.float32)
    gating = pl.pallas_call(
        _router_kernel,
        out_shape=jax.ShapeDtypeStruct((S, E), jnp.float32),
    )(expert, eps)
    flat_x = x.reshape(S, D)
    out = pl.pallas_call(
        _ffn_kernel,
        grid=(E, TB),
        in_specs=[
            pl.BlockSpec((S, E), lambda e, tb: (0, 0)),
            pl.BlockSpec((BT, D), lambda e, tb: (tb, 0)),
            pl.BlockSpec((1, D, H), lambda e, tb: (e, 0, 0)),
            pl.BlockSpec((1, 1, H), lambda e, tb: (e, 0, 0)),
            pl.BlockSpec((1, H, D), lambda e, tb: (e, 0, 0)),
            pl.BlockSpec((1, 1, D), lambda e, tb: (e, 0, 0)),
        ],
        out_specs=pl.BlockSpec((S, D), lambda e, tb: (0, 0)),
        out_shape=jax.ShapeDtypeStruct((S, D), jnp.float32),
        compiler_params=pltpu.CompilerParams(
            dimension_semantics=("arbitrary", "arbitrary")),
    )(gating, flat_x, W1, b1.reshape(E, 1, H), W2, b2.reshape(E, 1, D))
    return out.reshape(x.shape)


# speedup vs baseline: 1.5794x; 1.4147x over previous
import functools

import jax
import jax.numpy as jnp
from jax import lax
from jax.experimental import pallas as pl
from jax.experimental.pallas import tpu as pltpu
from jax.experimental.pallas import tpu_sc as plsc

S = 2048
D = 768
E = 8
K = 2
H = 4 * D
A = S * K
BT = 128
NBLK = A // BT
NSEG = NBLK + E - 1
NW = 32
CHW = A // NW
TKW = S // NW
CC = 256


def _router_kernel(zt_ref, epst_ref, pos_ref, counts_ref):
    z = zt_ref[...]
    eps = epst_ref[...]
    noisy = z + eps * jax.nn.softplus(z)
    idxe = lax.broadcasted_iota(jnp.int32, (E, S), 0)
    v0 = jnp.max(noisy, axis=0, keepdims=True)
    i0 = jnp.min(jnp.where(noisy == v0, idxe, E), axis=0, keepdims=True)
    m0 = idxe == i0
    masked = jnp.where(m0, -jnp.inf, noisy)
    v1 = jnp.max(masked, axis=0, keepdims=True)
    i1 = jnp.min(jnp.where(masked == v1, idxe, E), axis=0, keepdims=True)
    m1 = idxe == i1

    oh0 = m0.astype(jnp.float32)
    oh1 = m1.astype(jnp.float32)
    counts = jnp.sum(oh0 + oh1, axis=1, keepdims=True)
    tril = (lax.broadcasted_iota(jnp.int32, (E, E), 1)
            < lax.broadcasted_iota(jnp.int32, (E, E), 0)).astype(jnp.float32)
    c_hi = jnp.floor(counts * (1.0 / 64.0))
    c_lo = counts - c_hi * 64.0
    off = (jnp.dot(tril, c_hi, preferred_element_type=jnp.float32) * 64.0
           + jnp.dot(tril, c_lo, preferred_element_type=jnp.float32))
    counts_ref[...] = jnp.broadcast_to(counts, (E, 128)).astype(jnp.int32)

    up = (lax.broadcasted_iota(jnp.int32, (CC, CC), 0)
          < lax.broadcasted_iota(jnp.int32, (CC, CC), 1)).astype(jnp.float32)
    prefix = jnp.zeros((E, 1), jnp.float32)
    for slot, (oh, m) in enumerate(((oh0, m0), (oh1, m1))):
        for i in range(S // CC):
            blk = oh[:, i * CC:(i + 1) * CC]
            mblk = m[:, i * CC:(i + 1) * CC]
            rank = jnp.dot(blk, up, preferred_element_type=jnp.float32) + prefix
            dest = jnp.sum(jnp.where(mblk, rank + off, 0.0), axis=0, keepdims=True)
            pos_ref[slot:slot + 1, i * CC:(i + 1) * CC] = dest.astype(jnp.int32)
            prefix = prefix + jnp.sum(blk, axis=1, keepdims=True)


def _gate_kernel(z_ref, eps_ref, gateb_ref):
    z = z_ref[...]
    eps = eps_ref[...]
    noisy = z + eps * jax.nn.softplus(z)
    idxe = lax.broadcasted_iota(jnp.int32, (S, E), 1)
    v0 = jnp.max(noisy, axis=1, keepdims=True)
    i0 = jnp.min(jnp.where(noisy == v0, idxe, E), axis=1, keepdims=True)
    masked = jnp.where(idxe == i0, -jnp.inf, noisy)
    v1 = jnp.max(masked, axis=1, keepdims=True)
    t = jnp.exp(v1 - v0)
    g0 = 1.0 / (1.0 + t)
    g1 = t / (1.0 + t)
    gateb_ref[0:S, :] = jnp.broadcast_to(g0, (S, 16))
    gateb_ref[S:2 * S, :] = jnp.broadcast_to(g1, (S, 16))


def _gmm_kernel(bid_ref, gid_ref, rs_ref, re_ref,
                xg_ref, w1_ref, b1_ref, w2_ref, b2_ref, out_ref):
    s = pl.program_id(0)
    b = bid_ref[s]
    rs = rs_ref[s]
    re = re_ref[s]
    prev = bid_ref[jnp.maximum(s - 1, 0)]
    first = jnp.logical_or(s == 0, b != prev)

    @pl.when(first)
    def _():
        out_ref[...] = jnp.zeros_like(out_ref)

    @pl.when(re > rs)
    def _():
        x = xg_ref[...]
        h = jnp.maximum(
            jnp.dot(x, w1_ref[0], preferred_element_type=jnp.float32)
            + b1_ref[0], 0.0)
        o = jnp.dot(h, w2_ref[0], preferred_element_type=jnp.float32) + b2_ref[0]
        rows = lax.broadcasted_iota(jnp.int32, (BT, 1), 0)
        act = jnp.logical_and(rows >= rs, rows < re)
        out_ref[...] += jnp.where(act, o, 0.0)


def _dispatch_body(x_hbm, pos_hbm, xg_hbm, idx_v, xbuf, sem):
    c = lax.axis_index("c")
    sc = lax.axis_index("s")
    wid = sc * 2 + c
    tbase = (wid % 16) * CHW
    pltpu.sync_copy(x_hbm.at[pl.ds(tbase, CHW)], xbuf)
    pltpu.sync_copy(pos_hbm.at[pl.ds(wid * CHW, CHW)], idx_v)
    pltpu.async_copy(xbuf, xg_hbm.at[idx_v], sem).wait()


def _combine_body(y_hbm, pos_hbm, gateb_hbm, out_hbm,
                  i0_v, i1_v, g0_v, g1_v, buf0, buf1, sem):
    c = lax.axis_index("c")
    sc = lax.axis_index("s")
    wid = sc * 2 + c
    base = wid * TKW
    pltpu.sync_copy(pos_hbm.at[pl.ds(base, TKW)], i0_v)
    pltpu.sync_copy(pos_hbm.at[pl.ds(S + base, TKW)], i1_v)
    pltpu.sync_copy(gateb_hbm.at[pl.ds(base, TKW)], g0_v)
    pltpu.sync_copy(gateb_hbm.at[pl.ds(S + base, TKW)], g1_v)
    pltpu.async_copy(y_hbm.at[i0_v], buf0, sem).wait()
    pltpu.async_copy(y_hbm.at[i1_v], buf1, sem).wait()

    def row(r, carry):
        g0 = g0_v[r, pl.ds(0, 16)]
        g1 = g1_v[r, pl.ds(0, 16)]
        for j in range(D // 16):
            sl = pl.ds(j * 16, 16)
            buf0[r, sl] = buf0[r, sl] * g0 + buf1[r, sl] * g1
        return carry

    lax.fori_loop(0, TKW, row, 0)
    pltpu.sync_copy(buf0, out_hbm.at[pl.ds(base, TKW)])


def kernel(x, expert, W1, b1, W2, b2):
    eps = jax.random.normal(jax.random.key(42), expert.shape, dtype=jnp.float32)
    flat_x = x.reshape(S, D)

    pos, counts_b = pl.pallas_call(
        _router_kernel,
        out_shape=[
            jax.ShapeDtypeStruct((K, S), jnp.int32),
            jax.ShapeDtypeStruct((E, 128), jnp.int32),
        ],
    )(expert.T, eps.T)

    gateb = pl.pallas_call(
        _gate_kernel,
        out_shape=jax.ShapeDtypeStruct((K * S, 16), jnp.float32),
    )(expert, eps)

    counts = counts_b[:, 0]
    cum = jnp.cumsum(counts)
    starts = jnp.sort(jnp.concatenate(
        [jnp.arange(NBLK, dtype=jnp.int32) * BT, cum[:E - 1]]))
    ends = jnp.concatenate([starts[1:], jnp.full((1,), A, jnp.int32)])
    bid = jnp.minimum(starts // BT, NBLK - 1).astype(jnp.int32)
    gid = jnp.minimum(
        jnp.searchsorted(cum, starts, side="right"), E - 1).astype(jnp.int32)
    rs = jnp.clip(starts - bid * BT, 0, BT).astype(jnp.int32)
    re = jnp.clip(ends - bid * BT, 0, BT).astype(jnp.int32)

    mesh = plsc.VectorSubcoreMesh(core_axis_name="c", subcore_axis_name="s")
    pos_flat = pos.reshape(A)

    xg = pl.kernel(
        _dispatch_body,
        out_type=jax.ShapeDtypeStruct((A, D), jnp.float32),
        mesh=mesh,
        scratch_types=[
            pltpu.VMEM((CHW,), jnp.int32),
            pltpu.VMEM((CHW, D), jnp.float32),
            pltpu.SemaphoreType.DMA,
        ],
    )(flat_x, pos_flat)

    y = pl.pallas_call(
        _gmm_kernel,
        grid_spec=pltpu.PrefetchScalarGridSpec(
            num_scalar_prefetch=4,
            grid=(NSEG,),
            in_specs=[
                pl.BlockSpec((BT, D), lambda s, bid, gid, rs, re: (bid[s], 0)),
                pl.BlockSpec((1, D, H), lambda s, bid, gid, rs, re: (gid[s], 0, 0)),
                pl.BlockSpec((1, 1, H), lambda s, bid, gid, rs, re: (gid[s], 0, 0)),
                pl.BlockSpec((1, H, D), lambda s, bid, gid, rs, re: (gid[s], 0, 0)),
                pl.BlockSpec((1, 1, D), lambda s, bid, gid, rs, re: (gid[s], 0, 0)),
            ],
            out_specs=pl.BlockSpec(
                (BT, D), lambda s, bid, gid, rs, re: (bid[s], 0)),
        ),
        out_shape=jax.ShapeDtypeStruct((A, D), jnp.float32),
        compiler_params=pltpu.CompilerParams(
            dimension_semantics=("arbitrary",)),
    )(bid, gid, rs, re, xg, W1, b1.reshape(E, 1, H), W2, b2.reshape(E, 1, D))

    out = pl.kernel(
        _combine_body,
        out_type=jax.ShapeDtypeStruct((S, D), jnp.float32),
        mesh=mesh,
        scratch_types=[
            pltpu.VMEM((TKW,), jnp.int32),
            pltpu.VMEM((TKW,), jnp.int32),
            pltpu.VMEM((TKW, 16), jnp.float32),
            pltpu.VMEM((TKW, 16), jnp.float32),
            pltpu.VMEM((TKW, D), jnp.float32),
            pltpu.VMEM((TKW, D), jnp.float32),
            pltpu.SemaphoreType.DMA,
        ],
    )(y, pos_flat, gateb)

    return out.reshape(x.shape)


# speedup vs baseline: 1.6760x; 1.0612x over previous
import functools

import jax
import jax.numpy as jnp
from jax import lax
from jax.experimental import pallas as pl
from jax.experimental.pallas import tpu as pltpu
from jax.experimental.pallas import tpu_sc as plsc

S = 2048
D = 768
E = 8
K = 2
H = 4 * D
A = S * K
BT = 256
NBLK = A // BT
NSEG = NBLK + E - 1
NW = 32
CHW = A // NW
TKW = S // NW
CC = 256


def _router_kernel(zt_ref, epst_ref, pos_ref, counts_ref):
    z = zt_ref[...]
    eps = epst_ref[...]
    noisy = z + eps * jax.nn.softplus(z)
    idxe = lax.broadcasted_iota(jnp.int32, (E, S), 0)
    v0 = jnp.max(noisy, axis=0, keepdims=True)
    i0 = jnp.min(jnp.where(noisy == v0, idxe, E), axis=0, keepdims=True)
    m0 = idxe == i0
    masked = jnp.where(m0, -jnp.inf, noisy)
    v1 = jnp.max(masked, axis=0, keepdims=True)
    i1 = jnp.min(jnp.where(masked == v1, idxe, E), axis=0, keepdims=True)
    m1 = idxe == i1

    oh0 = m0.astype(jnp.float32)
    oh1 = m1.astype(jnp.float32)
    counts = jnp.sum(oh0 + oh1, axis=1, keepdims=True)
    tril = (lax.broadcasted_iota(jnp.int32, (E, E), 1)
            < lax.broadcasted_iota(jnp.int32, (E, E), 0)).astype(jnp.float32)
    c_hi = jnp.floor(counts * (1.0 / 64.0))
    c_lo = counts - c_hi * 64.0
    off = (jnp.dot(tril, c_hi, preferred_element_type=jnp.float32) * 64.0
           + jnp.dot(tril, c_lo, preferred_element_type=jnp.float32))
    counts_ref[...] = jnp.broadcast_to(counts, (E, 128)).astype(jnp.int32)

    up = (lax.broadcasted_iota(jnp.int32, (CC, CC), 0)
          < lax.broadcasted_iota(jnp.int32, (CC, CC), 1)).astype(jnp.float32)
    prefix = jnp.zeros((E, 1), jnp.float32)
    for slot, (oh, m) in enumerate(((oh0, m0), (oh1, m1))):
        for i in range(S // CC):
            blk = oh[:, i * CC:(i + 1) * CC]
            mblk = m[:, i * CC:(i + 1) * CC]
            rank = jnp.dot(blk, up, preferred_element_type=jnp.float32) + prefix
            dest = jnp.sum(jnp.where(mblk, rank + off, 0.0), axis=0, keepdims=True)
            pos_ref[slot:slot + 1, i * CC:(i + 1) * CC] = dest.astype(jnp.int32)
            prefix = prefix + jnp.sum(blk, axis=1, keepdims=True)


def _gate_kernel(z_ref, eps_ref, gateb_ref):
    z = z_ref[...]
    eps = eps_ref[...]
    noisy = z + eps * jax.nn.softplus(z)
    idxe = lax.broadcasted_iota(jnp.int32, (S, E), 1)
    v0 = jnp.max(noisy, axis=1, keepdims=True)
    i0 = jnp.min(jnp.where(noisy == v0, idxe, E), axis=1, keepdims=True)
    masked = jnp.where(idxe == i0, -jnp.inf, noisy)
    v1 = jnp.max(masked, axis=1, keepdims=True)
    t = jnp.exp(v1 - v0)
    g0 = 1.0 / (1.0 + t)
    g1 = t / (1.0 + t)
    gateb_ref[0:S, :] = jnp.broadcast_to(g0, (S, 16))
    gateb_ref[S:2 * S, :] = jnp.broadcast_to(g1, (S, 16))


def _gmm_kernel(bid_ref, gid_ref, rs_ref, re_ref,
                xg_ref, w1_ref, b1_ref, w2_ref, b2_ref, out_ref):
    s = pl.program_id(0)
    b = bid_ref[s]
    rs = rs_ref[s]
    re = re_ref[s]
    prev = bid_ref[jnp.maximum(s - 1, 0)]
    first = jnp.logical_or(s == 0, b != prev)

    @pl.when(first)
    def _():
        out_ref[...] = jnp.zeros_like(out_ref)

    @pl.when(re > rs)
    def _():
        x = xg_ref[...]
        h = jnp.maximum(
            jnp.dot(x, w1_ref[0], preferred_element_type=jnp.float32)
            + b1_ref[0], 0.0)
        o = jnp.dot(h, w2_ref[0], preferred_element_type=jnp.float32) + b2_ref[0]
        rows = lax.broadcasted_iota(jnp.int32, (BT, 1), 0)
        act = jnp.logical_and(rows >= rs, rows < re)
        out_ref[...] += jnp.where(act, o, 0.0)


def _dispatch_body(x_hbm, pos_hbm, xg_hbm, idx_v, xbuf, sem):
    c = lax.axis_index("c")
    sc = lax.axis_index("s")
    wid = sc * 2 + c
    tbase = (wid % 16) * CHW
    pltpu.sync_copy(x_hbm.at[pl.ds(tbase, CHW)], xbuf)
    pltpu.sync_copy(pos_hbm.at[pl.ds(wid * CHW, CHW)], idx_v)
    pltpu.async_copy(xbuf, xg_hbm.at[idx_v], sem).wait()


def _combine_body(y_hbm, pos_hbm, gateb_hbm, out_hbm,
                  i0_v, i1_v, g0_v, g1_v, buf0, buf1, sem):
    c = lax.axis_index("c")
    sc = lax.axis_index("s")
    wid = sc * 2 + c
    base = wid * TKW
    pltpu.sync_copy(pos_hbm.at[pl.ds(base, TKW)], i0_v)
    pltpu.sync_copy(pos_hbm.at[pl.ds(S + base, TKW)], i1_v)
    pltpu.sync_copy(gateb_hbm.at[pl.ds(base, TKW)], g0_v)
    pltpu.sync_copy(gateb_hbm.at[pl.ds(S + base, TKW)], g1_v)
    pltpu.async_copy(y_hbm.at[i0_v], buf0, sem).wait()
    pltpu.async_copy(y_hbm.at[i1_v], buf1, sem).wait()

    def row(r, carry):
        g0 = g0_v[r, pl.ds(0, 16)]
        g1 = g1_v[r, pl.ds(0, 16)]
        for j in range(D // 16):
            sl = pl.ds(j * 16, 16)
            buf0[r, sl] = buf0[r, sl] * g0 + buf1[r, sl] * g1
        return carry

    lax.fori_loop(0, TKW, row, 0)
    pltpu.sync_copy(buf0, out_hbm.at[pl.ds(base, TKW)])


def kernel(x, expert, W1, b1, W2, b2):
    eps = jax.random.normal(jax.random.key(42), expert.shape, dtype=jnp.float32)
    flat_x = x.reshape(S, D)

    pos, counts_b = pl.pallas_call(
        _router_kernel,
        out_shape=[
            jax.ShapeDtypeStruct((K, S), jnp.int32),
            jax.ShapeDtypeStruct((E, 128), jnp.int32),
        ],
    )(expert.T, eps.T)

    gateb = pl.pallas_call(
        _gate_kernel,
        out_shape=jax.ShapeDtypeStruct((K * S, 16), jnp.float32),
    )(expert, eps)

    counts = counts_b[:, 0]
    cum = jnp.cumsum(counts)
    starts = jnp.sort(jnp.concatenate(
        [jnp.arange(NBLK, dtype=jnp.int32) * BT, cum[:E - 1]]))
    ends = jnp.concatenate([starts[1:], jnp.full((1,), A, jnp.int32)])
    bid = jnp.minimum(starts // BT, NBLK - 1).astype(jnp.int32)
    gid = jnp.minimum(
        jnp.searchsorted(cum, starts, side="right"), E - 1).astype(jnp.int32)
    rs = jnp.clip(starts - bid * BT, 0, BT).astype(jnp.int32)
    re = jnp.clip(ends - bid * BT, 0, BT).astype(jnp.int32)

    mesh = plsc.VectorSubcoreMesh(core_axis_name="c", subcore_axis_name="s")
    pos_flat = pos.reshape(A)

    xg = pl.kernel(
        _dispatch_body,
        out_type=jax.ShapeDtypeStruct((A, D), jnp.float32),
        mesh=mesh,
        scratch_types=[
            pltpu.VMEM((CHW,), jnp.int32),
            pltpu.VMEM((CHW, D), jnp.float32),
            pltpu.SemaphoreType.DMA,
        ],
    )(flat_x, pos_flat)

    y = pl.pallas_call(
        _gmm_kernel,
        grid_spec=pltpu.PrefetchScalarGridSpec(
            num_scalar_prefetch=4,
            grid=(NSEG,),
            in_specs=[
                pl.BlockSpec((BT, D), lambda s, bid, gid, rs, re: (bid[s], 0)),
                pl.BlockSpec((1, D, H), lambda s, bid, gid, rs, re: (gid[s], 0, 0)),
                pl.BlockSpec((1, 1, H), lambda s, bid, gid, rs, re: (gid[s], 0, 0)),
                pl.BlockSpec((1, H, D), lambda s, bid, gid, rs, re: (gid[s], 0, 0)),
                pl.BlockSpec((1, 1, D), lambda s, bid, gid, rs, re: (gid[s], 0, 0)),
            ],
            out_specs=pl.BlockSpec(
                (BT, D), lambda s, bid, gid, rs, re: (bid[s], 0)),
        ),
        out_shape=jax.ShapeDtypeStruct((A, D), jnp.float32),
        compiler_params=pltpu.CompilerParams(
            dimension_semantics=("arbitrary",)),
    )(bid, gid, rs, re, xg, W1, b1.reshape(E, 1, H), W2, b2.reshape(E, 1, D))

    out = pl.kernel(
        _combine_body,
        out_type=jax.ShapeDtypeStruct((S, D), jnp.float32),
        mesh=mesh,
        scratch_types=[
            pltpu.VMEM((TKW,), jnp.int32),
            pltpu.VMEM((TKW,), jnp.int32),
            pltpu.VMEM((TKW, 16), jnp.float32),
            pltpu.VMEM((TKW, 16), jnp.float32),
            pltpu.VMEM((TKW, D), jnp.float32),
            pltpu.VMEM((TKW, D), jnp.float32),
            pltpu.SemaphoreType.DMA,
        ],
    )(y, pos_flat, gateb)

    return out.reshape(x.shape)
